# Initial kernel scaffold; baseline (speedup 1.0000x reference)
import jax
import jax.numpy as jnp
from jax.experimental import pallas as pl


def kernel(s_hist, s, r, ent_embeds, rel_embeds, W, b):
    raise NotImplementedError("write your pallas kernel here")


# speedup vs baseline: 9.0822x; 9.0822x over previous
import functools

import jax
import jax.numpy as jnp
from jax import lax
from jax.experimental import pallas as pl
from jax.experimental.pallas import tpu as pltpu
from jax.experimental.pallas import tpu_sc as plsc

B, T, K, H = 1024, 10, 16, 256
NW = 32
GROUPS = B * T
GPW = GROUPS // NW
GB = 8
RB = GB * K
NBATCH = GPW // GB
SPW = B // NW
LANES = 16
C = H // LANES


def _sc_gather_sum(hist, s, r, ent, rel):
    mesh = plsc.VectorSubcoreMesh(core_axis_name="c", subcore_axis_name="s")

    @functools.partial(
        pl.kernel,
        mesh=mesh,
        out_type=[
            jax.ShapeDtypeStruct((GROUPS, H), jnp.float32),
            jax.ShapeDtypeStruct((B, H), jnp.float32),
            jax.ShapeDtypeStruct((B, H), jnp.float32),
        ],
        scratch_types=[
            pltpu.VMEM((GPW * K,), jnp.int32),
            pltpu.VMEM((RB, H), jnp.float32),
            pltpu.VMEM((RB, H), jnp.float32),
            pltpu.VMEM((GB, H), jnp.float32),
            pltpu.VMEM((SPW,), jnp.int32),
            pltpu.VMEM((SPW, H), jnp.float32),
            pltpu.SemaphoreType.DMA,
            pltpu.SemaphoreType.DMA,
        ],
    )
    def k(hist_hbm, s_hbm, r_hbm, ent_hbm, rel_hbm,
          sums_hbm, se_hbm, re_hbm,
          idx_v, rows0, rows1, sums_v, sidx_v, srows_v, sem0, sem1):
        cid = lax.axis_index("c")
        sid = lax.axis_index("s")
        wid = sid * 2 + cid
        base_g = wid * GPW

        pltpu.sync_copy(hist_hbm.at[pl.ds(base_g * K, GPW * K)], idx_v)

        def start_gather(bi, rows_ref, sem):
            pltpu.make_async_copy(
                ent_hbm.at[idx_v.at[pl.ds(bi * RB, RB)]], rows_ref, sem
            ).start()

        def finish_batch(bi, rows_ref, sem):
            pltpu.make_async_copy(
                ent_hbm.at[idx_v.at[pl.ds(bi * RB, RB)]], rows_ref, sem
            ).wait()
            for g in range(GB):
                def row_body(rr, acc, g=g):
                    return tuple(
                        acc[c] + rows_ref[g * K + rr, pl.ds(c * LANES, LANES)]
                        for c in range(C)
                    )
                acc = tuple(
                    rows_ref[g * K, pl.ds(c * LANES, LANES)] for c in range(C)
                )
                acc = lax.fori_loop(1, K, row_body, acc)
                for c in range(C):
                    sums_v[g, pl.ds(c * LANES, LANES)] = acc[c]
            pltpu.sync_copy(sums_v, sums_hbm.at[pl.ds(base_g + bi * GB, GB)])

        start_gather(0, rows0, sem0)

        def outer(j, carry):
            start_gather(2 * j + 1, rows1, sem1)
            finish_batch(2 * j, rows0, sem0)

            @pl.when(j < NBATCH // 2 - 1)
            def _():
                start_gather(2 * j + 2, rows0, sem0)

            finish_batch(2 * j + 1, rows1, sem1)
            return carry

        lax.fori_loop(0, NBATCH // 2, outer, 0)

        sb = wid * SPW
        pltpu.sync_copy(s_hbm.at[pl.ds(sb, SPW)], sidx_v)
        pltpu.async_copy(ent_hbm.at[sidx_v], srows_v, sem0).wait()
        pltpu.sync_copy(srows_v, se_hbm.at[pl.ds(sb, SPW)])
        pltpu.sync_copy(r_hbm.at[pl.ds(sb, SPW)], sidx_v)
        pltpu.async_copy(rel_hbm.at[sidx_v], srows_v, sem0).wait()
        pltpu.sync_copy(srows_v, re_hbm.at[pl.ds(sb, SPW)])

    return k(hist, s, r, ent, rel)


def _tc_finish(sums3, s_e, r_e, W, b2):
    BB = 64

    def body(sums_ref, se_ref, re_ref, w_ref, b_ref, out_ref):
        x = sums_ref[...].reshape(BB * T, H) * (1.0 / K)
        y = jnp.dot(x, w_ref[...], preferred_element_type=jnp.float32)
        y = jnp.maximum(y + b_ref[...], 0.0)
        out_ref[:, :, 0:H] = y.reshape(BB, T, H)
        out_ref[:, :, H:2 * H] = jnp.broadcast_to(
            se_ref[...][:, None, :], (BB, T, H))
        out_ref[:, :, 2 * H:3 * H] = jnp.broadcast_to(
            re_ref[...][:, None, :], (BB, T, H))

    return pl.pallas_call(
        body,
        grid=(B // BB,),
        in_specs=[
            pl.BlockSpec((BB, T, H), lambda i: (i, 0, 0)),
            pl.BlockSpec((BB, H), lambda i: (i, 0)),
            pl.BlockSpec((BB, H), lambda i: (i, 0)),
            pl.BlockSpec((H, H), lambda i: (0, 0)),
            pl.BlockSpec((1, H), lambda i: (0, 0)),
        ],
        out_specs=pl.BlockSpec((BB, T, 3 * H), lambda i: (i, 0, 0)),
        out_shape=jax.ShapeDtypeStruct((B, T, 3 * H), jnp.float32),
    )(sums3, s_e, r_e, W, b2)


def kernel(s_hist, s, r, ent_embeds, rel_embeds, W, b):
    hist = s_hist.reshape(-1).astype(jnp.int32)
    sums, s_e, r_e = _sc_gather_sum(
        hist, s.astype(jnp.int32), r.astype(jnp.int32), ent_embeds, rel_embeds)
    return _tc_finish(
        sums.reshape(B, T, H), s_e, r_e, W, b.reshape(1, H))
